# baseline (speedup 1.0000x reference)
import functools

import jax
import jax.numpy as jnp
from jax import lax
from jax.experimental import pallas as pl
from jax.experimental.pallas import tpu as pltpu
from jax.experimental.pallas import tpu_sc as plsc

_TOP_P = 0.8
_B = 64
_V = 100000
_L = 16
_NW = 32
_ROWS_PER_W = _B // _NW
_NCHUNK = _V // _L
_UNROLL = 10
_HBINS = 1024
_HWORDS = _L * _HBINS


def _lane_iota():
    return lax.iota(jnp.int32, _L)


def _process_row(row_v, hist_v):
    lanes = _lane_iota()
    lane_base = lanes * _HBINS

    def mx_body(i, acc):
        for u in range(_UNROLL):
            acc = jnp.maximum(acc, row_v[pl.ds(i * _L * _UNROLL + u * _L, _L)])
        return acc
    acc = lax.fori_loop(0, _NCHUNK // _UNROLL, mx_body,
                        jnp.full((_L,), -jnp.inf, jnp.float32))
    m = jnp.max(acc)

    def zero_hist():
        zeros = jnp.zeros((_L,), jnp.float32)
        def z_body(i, carry):
            for u in range(8):
                hist_v[pl.ds(i * 8 * _L + u * _L, _L)] = zeros
            return carry
        lax.fori_loop(0, _HWORDS // (8 * _L), z_body, 0)

    zero_hist()

    def e_body(i, zacc):
        for u in range(_UNROLL):
            sl = pl.ds(i * _L * _UNROLL + u * _L, _L)
            v = jnp.exp(row_v[sl] - m)
            row_v[sl] = v
            zacc = zacc + v
            bits = plsc.bitcast(v, jnp.int32)
            idx = lane_base + (bits >> 20)
            plsc.addupdate_scatter(hist_v, [idx], v)
        return zacc
    zacc = lax.fori_loop(0, _NCHUNK // _UNROLL, e_body, jnp.zeros((_L,), jnp.float32))
    z = jnp.sum(zacc)
    target = _TOP_P * z

    def combined(c):
        comb = hist_v[pl.ds(c * _L, _L)]
        for l in range(1, _L):
            comb = comb + hist_v[pl.ds(l * _HBINS + c * _L, _L)]
        return comb

    def scan_level(t):
        def w_cond(st):
            c, r, done = st
            return jnp.logical_not(done)

        def w_body(st):
            c, r, done = st
            tot = jnp.sum(combined(c))
            new = r + tot
            cross = jnp.logical_or(new > t, c == 0)
            return (jnp.where(cross, c, c - 1),
                    jnp.where(cross, r, new),
                    cross)

        c, r, _ = lax.while_loop(w_cond, w_body,
                                 (jnp.int32(_HBINS // _L - 1), jnp.float32(0.0),
                                  jnp.bool_(False)))
        comb = combined(c)
        pre = plsc.cumsum(comb)
        tot = jnp.sum(comb)
        above = r + (tot - pre)
        ok = above <= t
        j = jnp.broadcast_to(plsc.all_reduce_ffs(ok), (_L,))
        pre_j = jnp.sum(jnp.where(lanes == j, pre, 0.0))
        h_j = jnp.sum(jnp.where(lanes == j, comb, 0.0))
        m_above = r + (tot - pre_j)
        b = c * _L + j
        return b, m_above, h_j

    b1, m1, _ = scan_level(target)
    t2 = target - m1

    zero_hist()

    def h2_body(i, carry):
        for u in range(_UNROLL):
            sl = pl.ds(i * _L * _UNROLL + u * _L, _L)
            v = row_v[sl]
            bits = plsc.bitcast(v, jnp.int32)
            sel = (bits >> 20) == b1
            idx = lane_base + ((bits >> 10) & 0x3FF)
            plsc.addupdate_scatter(hist_v, [idx], v, mask=sel)
        return carry
    lax.fori_loop(0, _NCHUNK // _UNROLL, h2_body, 0)
    b2, m2, _ = scan_level(t2)
    t3 = t2 - m2

    zero_hist()
    hi = b1 * 1024 + b2

    def h3_body(i, carry):
        for u in range(_UNROLL):
            sl = pl.ds(i * _L * _UNROLL + u * _L, _L)
            v = row_v[sl]
            bits = plsc.bitcast(v, jnp.int32)
            sel = (bits >> 10) == hi
            idx = lane_base + (bits & 0x3FF)
            plsc.addupdate_scatter(hist_v, [idx], v, mask=sel)
        return carry
    lax.fori_loop(0, _NCHUNK // _UNROLL, h3_body, 0)
    b3, m3, h3 = scan_level(t3)

    kstar = (b1 << 20) | (b2 << 10) | b3
    s = m1 + m2 + m3 + h3
    s_vec = jnp.broadcast_to(s, (_L,))
    r0 = plsc.bitcast(jnp.broadcast_to(jnp.int32(0x7EF477D5), (_L,))
                      - plsc.bitcast(s_vec, jnp.int32), jnp.float32)
    for _ in range(4):
        r0 = r0 * (2.0 - s_vec * r0)
    rs = r0

    def w_body(i, carry):
        for u in range(_UNROLL):
            sl = pl.ds(i * _L * _UNROLL + u * _L, _L)
            v = row_v[sl]
            bits = plsc.bitcast(v, jnp.int32)
            keep = bits >= kstar
            row_v[sl] = jnp.where(keep, v * rs, 0.0)
        return carry
    lax.fori_loop(0, _NCHUNK // _UNROLL, w_body, 0)


def _sc_body(logits_hbm, out_hbm, row_v, hist_v):
    wid = lax.axis_index("s") * 2 + lax.axis_index("c")
    for rb in range(_ROWS_PER_W):
        r = wid * _ROWS_PER_W + rb
        pltpu.sync_copy(logits_hbm.at[r], row_v)
        _process_row(row_v, hist_v)
        pltpu.sync_copy(row_v, out_hbm.at[r])


def kernel(logits):
    f = functools.partial(
        pl.kernel,
        out_type=jax.ShapeDtypeStruct((_B, _V), jnp.float32),
        mesh=plsc.VectorSubcoreMesh(core_axis_name="c", subcore_axis_name="s"),
        scratch_types=[
            pltpu.VMEM((_V,), jnp.float32),
            pltpu.VMEM((_HWORDS,), jnp.float32),
        ],
        compiler_params=pltpu.CompilerParams(needs_layout_passes=False),
    )(_sc_body)
    return f(logits)


# speedup vs baseline: 4.0623x; 4.0623x over previous
import functools

import jax
import jax.numpy as jnp
from jax import lax
from jax.experimental import pallas as pl
from jax.experimental.pallas import tpu as pltpu
from jax.experimental.pallas import tpu_sc as plsc

_TOP_P = 0.8
_B = 64
_V = 100000
_L = 16
_NW = 32
_ROWS_PER_W = _B // _NW
_UNROLL = 10
_HBINS = 1024
_HWORDS = _L * _HBINS


def _process_row(row_v, hist_v):
    lanes = lax.iota(jnp.int32, _L)

    @plsc.parallel_loop(0, _V, _L, unroll=_UNROLL,
                        carry=jnp.full((_L,), -jnp.inf, jnp.float32))
    def mx_loop(i, acc):
        return jnp.maximum(acc, row_v[pl.ds(i, _L)])
    m = jnp.max(mx_loop)

    def zero_hist():
        zeros = jnp.zeros((_L,), jnp.float32)

        @plsc.parallel_loop(0, _HWORDS, _L, unroll=8)
        def z_loop(i):
            hist_v[pl.ds(i, _L)] = zeros

    zero_hist()

    @plsc.parallel_loop(0, _V, _L, unroll=_UNROLL,
                        carry=jnp.zeros((_L,), jnp.float32))
    def e_loop(i, zacc):
        v = jnp.exp(row_v[pl.ds(i, _L)] - m)
        row_v[pl.ds(i, _L)] = v
        bits = plsc.bitcast(v, jnp.int32)
        idx = (bits >> 20) * _L + lanes
        plsc.addupdate_scatter(hist_v, [idx], v)
        return zacc + v
    z = jnp.sum(e_loop)
    target = _TOP_P * z

    def scan_level(t):
        def chunk_vec(c):
            acc = hist_v[pl.ds(c * (_L * _L), _L)]
            for w in range(1, _L):
                acc = acc + hist_v[pl.ds(c * (_L * _L) + w * _L, _L)]
            return acc

        def a_cond(st):
            return jnp.logical_not(st[2])

        def a_body(st):
            c, r, done = st
            new = r + jnp.sum(chunk_vec(c))
            cross = jnp.logical_or(new > t, c == 0)
            return (jnp.where(cross, c, c - 1), jnp.where(cross, r, new), cross)

        c, r, _ = lax.while_loop(
            a_cond, a_body,
            (jnp.int32(_HBINS // _L - 1), jnp.float32(0.0), jnp.bool_(False)))

        def b_cond(st):
            return jnp.logical_not(st[3])

        def b_body(st):
            w, r2, _, done = st
            h = jnp.sum(hist_v[pl.ds((c * _L + w) * _L, _L)])
            cross = jnp.logical_or(r2 + h > t, w == 0)
            return (jnp.where(cross, w, w - 1), jnp.where(cross, r2, r2 + h),
                    h, cross)

        w, r2, h_b, _ = lax.while_loop(
            b_cond, b_body,
            (jnp.int32(_L - 1), r, jnp.float32(0.0), jnp.bool_(False)))
        b = c * _L + w
        return b, r2, h_b

    b1, m1, _ = scan_level(target)
    t2 = target - m1

    zero_hist()

    @plsc.parallel_loop(0, _V, _L, unroll=_UNROLL)
    def h2_loop(i):
        v = row_v[pl.ds(i, _L)]
        bits = plsc.bitcast(v, jnp.int32)
        sel = (bits >> 20) == b1
        idx = ((bits >> 10) & 0x3FF) * _L + lanes
        plsc.addupdate_scatter(hist_v, [idx], v, mask=sel)

    b2, m2, _ = scan_level(t2)
    t3 = t2 - m2

    zero_hist()
    hi = b1 * 1024 + b2

    @plsc.parallel_loop(0, _V, _L, unroll=_UNROLL)
    def h3_loop(i):
        v = row_v[pl.ds(i, _L)]
        bits = plsc.bitcast(v, jnp.int32)
        sel = (bits >> 10) == hi
        idx = (bits & 0x3FF) * _L + lanes
        plsc.addupdate_scatter(hist_v, [idx], v, mask=sel)

    b3, m3, h3 = scan_level(t3)

    kstar = (b1 << 20) | (b2 << 10) | b3
    s = m1 + m2 + m3 + h3
    s_vec = jnp.broadcast_to(s, (_L,))
    r0 = plsc.bitcast(jnp.broadcast_to(jnp.int32(0x7EF477D5), (_L,))
                      - plsc.bitcast(s_vec, jnp.int32), jnp.float32)
    for _ in range(4):
        r0 = r0 * (2.0 - s_vec * r0)
    rs = r0

    @plsc.parallel_loop(0, _V, _L, unroll=_UNROLL)
    def w_loop(i):
        v = row_v[pl.ds(i, _L)]
        keep = plsc.bitcast(v, jnp.int32) >= kstar
        row_v[pl.ds(i, _L)] = jnp.where(keep, v * rs, 0.0)


def _sc_body(logits_hbm, out_hbm, row_v, hist_v):
    wid = lax.axis_index("s") * 2 + lax.axis_index("c")
    for rb in range(_ROWS_PER_W):
        r = wid * _ROWS_PER_W + rb
        pltpu.sync_copy(logits_hbm.at[r], row_v)
        _process_row(row_v, hist_v)
        pltpu.sync_copy(row_v, out_hbm.at[r])


def kernel(logits):
    f = functools.partial(
        pl.kernel,
        out_type=jax.ShapeDtypeStruct((_B, _V), jnp.float32),
        mesh=plsc.VectorSubcoreMesh(core_axis_name="c", subcore_axis_name="s"),
        scratch_types=[
            pltpu.VMEM((_V,), jnp.float32),
            pltpu.VMEM((_HWORDS,), jnp.float32),
        ],
        compiler_params=pltpu.CompilerParams(needs_layout_passes=False),
    )(_sc_body)
    return f(logits)
